# Initial kernel scaffold; baseline (speedup 1.0000x reference)
import jax
import jax.numpy as jnp
from jax.experimental import pallas as pl


def kernel(x, w1, b1, w2, b2, w3, b3, gamma, beta):
    raise NotImplementedError("write your pallas kernel here")


# speedup vs baseline: 3.0006x; 3.0006x over previous
import jax
import jax.numpy as jnp
from jax.experimental import pallas as pl
from jax.experimental.pallas import tpu as pltpu

KN = 5
EPS = 1e-5
PAD = (KN - 1) // 2


def _conv1_stats_kernel(xp_ref, w1_ref, b1_ref, h_ref, stats_ref, *, H, W, C):
    i = pl.program_id(1)

    @pl.when(i == 0)
    def _init():
        stats_ref[...] = jnp.zeros_like(stats_ref)

    HW = H * W
    XR = HW + 2 * PAD * W
    xp = xp_ref[0]

    w_co = jax.lax.broadcasted_iota(jnp.int32, (XR, C), 0) % W
    cols = []
    for kw in range(KN):
        sl = xp[kw:kw + XR]
        lo, hi = PAD - kw, W + PAD - kw
        if lo > 0:
            sl = jnp.where(w_co >= lo, sl, jnp.bfloat16(0))
        if hi < W:
            sl = jnp.where(w_co < hi, sl, jnp.bfloat16(0))
        cols.append(sl)
    xc = jnp.concatenate(cols, axis=1)

    acc = jnp.zeros((HW, C), jnp.float32)
    for kh in range(KN):
        acc = acc + jnp.dot(xc[kh * W:kh * W + HW],
                            w1_ref[kh * KN * C:(kh * KN + KN) * C],
                            preferred_element_type=jnp.float32)
    h = acc + b1_ref[...]
    h_ref[0] = h.astype(jnp.bfloat16)

    stats_ref[0, 0:1, :] += jnp.sum(h, axis=0, keepdims=True)
    stats_ref[0, 1:2, :] += jnp.sum(h * h, axis=0, keepdims=True)


def _apply_kernel(h_ref, xp_ref, w2_ref, b2_ref, w3_ref, b3_ref, o_ref, *, H, W):
    base = PAD * W + PAD
    h = h_ref[0]
    a = jnp.dot(h, w2_ref[...], preferred_element_type=jnp.float32) + b2_ref[...]
    a = jnp.maximum(a, 0.0).astype(jnp.bfloat16)
    o = jnp.dot(a, w3_ref[...], preferred_element_type=jnp.float32) + b3_ref[...]
    res = xp_ref[0, base:base + H * W, :].astype(jnp.float32)
    o_ref[0] = o + res


def kernel(x, w1, b1, w2, b2, w3, b3, gamma, beta):
    N, C, H, W = x.shape
    HW = H * W
    NHW = N * HW
    C2 = 2 * C
    KK = KN * KN

    pad_top = PAD * W + PAD
    HWP = -(-(HW + 2 * pad_top + 2 * PAD) // 8) * 8
    x_t = jnp.transpose(x.reshape(N, C, HW), (0, 2, 1)).astype(jnp.bfloat16)
    xp = jnp.pad(x_t, ((0, 0), (pad_top, HWP - HW - pad_top), (0, 0)))

    w1col = jnp.transpose(w1, (2, 3, 1, 0)).reshape(KK * C, C).astype(jnp.bfloat16)
    b1r = b1.reshape(1, C)

    cores = 2 if N % 2 == 0 else 1
    steps = N // cores
    h_raw, stats = pl.pallas_call(
        lambda *a: _conv1_stats_kernel(*a, H=H, W=W, C=C),
        out_shape=(jax.ShapeDtypeStruct((N, HW, C), jnp.bfloat16),
                   jax.ShapeDtypeStruct((cores, 2, C), jnp.float32)),
        grid=(cores, steps),
        in_specs=[
            pl.BlockSpec((1, HWP, C), lambda c, i: (c * steps + i, 0, 0)),
            pl.BlockSpec((KK * C, C), lambda c, i: (0, 0)),
            pl.BlockSpec((1, C), lambda c, i: (0, 0)),
        ],
        out_specs=(
            pl.BlockSpec((1, HW, C), lambda c, i: (c * steps + i, 0, 0)),
            pl.BlockSpec((1, 2, C), lambda c, i: (c, 0, 0)),
        ),
        compiler_params=pltpu.CompilerParams(
            dimension_semantics=("parallel", "arbitrary"),
            vmem_limit_bytes=64 * 1024 * 1024),
    )(xp, w1col, b1r)

    s = jnp.sum(stats, axis=0)
    mean = s[0] / NHW
    var = jnp.maximum(s[1] / NHW - mean * mean, 0.0)
    scale = gamma * jax.lax.rsqrt(var + EPS)
    shift = beta - mean * scale
    w2m = jnp.transpose(w2[:, :, 0, 0], (1, 0))
    w2f = (w2m * scale[:, None]).astype(jnp.bfloat16)
    b2f = (b2.reshape(1, C2) + shift.reshape(1, C) @ w2m).astype(jnp.float32)
    w3m = jnp.transpose(w3[:, :, 0, 0], (1, 0)).astype(jnp.bfloat16)
    b3r = b3.reshape(1, C)

    out = pl.pallas_call(
        lambda *a: _apply_kernel(*a, H=H, W=W),
        out_shape=jax.ShapeDtypeStruct((N, HW, C), jnp.float32),
        grid=(N,),
        in_specs=[
            pl.BlockSpec((1, HW, C), lambda i: (i, 0, 0)),
            pl.BlockSpec((1, HWP, C), lambda i: (i, 0, 0)),
            pl.BlockSpec((C, C2), lambda i: (0, 0)),
            pl.BlockSpec((1, C2), lambda i: (0, 0)),
            pl.BlockSpec((C2, C), lambda i: (0, 0)),
            pl.BlockSpec((1, C), lambda i: (0, 0)),
        ],
        out_specs=pl.BlockSpec((1, HW, C), lambda i: (i, 0, 0)),
        compiler_params=pltpu.CompilerParams(
            dimension_semantics=("parallel",),
            vmem_limit_bytes=64 * 1024 * 1024),
    )(h_raw, xp, w2f, b2f, w3m, b3r)

    return jnp.transpose(out, (0, 2, 1)).reshape(N, C, H, W)
